# baseline (speedup 1.0000x reference)
import jax
import jax.numpy as jnp
from jax import lax
from jax.experimental import pallas as pl
from jax.experimental.pallas import tpu as pltpu

_N = 4096
_K = 1024
_T = 4.0
_BLK = 512


def _mask_prologue(alpha_ref, mask_ref):
    @pl.when(pl.program_id(0) == 0)
    def _():
        u = alpha_ref[...] * (1.0 / _T)
        mx = jnp.max(u)
        e = jnp.exp(u - mx)
        s = jnp.sum(e)
        bits = jax.lax.bitcast_convert_type(e, jnp.int32)

        def vstep(_, lohi):
            lo, hi = lohi
            mid = lo + (hi - lo) // 2
            ok = jnp.sum((bits >= mid).astype(jnp.int32)) >= _K
            return jnp.where(ok, mid, lo), jnp.where(ok, hi, mid)

        tb, _ = lax.fori_loop(0, 30, vstep,
                              (jnp.int32(0), jnp.int32(1 << 30)))
        n_ge = jnp.sum((bits >= tb).astype(jnp.int32))
        tie = bits == tb
        n_eq = jnp.sum(tie.astype(jnp.int32))
        need = _K - (n_ge - n_eq)

        idx = lax.broadcasted_iota(jnp.int32, (1, _N), 1)

        def istep(_, lohi):
            lo, hi = lohi
            mid = lo + (hi - lo) // 2
            cnt = jnp.sum(jnp.where(tie & (idx >= mid), 1, 0))
            ok = cnt >= need
            return jnp.where(ok, mid, lo), jnp.where(ok, hi, mid)

        c, _ = lax.fori_loop(0, 12, istep,
                             (jnp.int32(0), jnp.int32(_N)))
        keep = (bits > tb) | (tie & (idx >= c))
        mask_ref[...] = jnp.where(keep, e / s, 0.0)


def _fused_body(alpha_ref, x_ref, bias_ref, o_ref, mask_ref):
    _mask_prologue(alpha_ref, mask_ref)
    o_ref[...] = x_ref[...] * mask_ref[...] + bias_ref[...]


def _fused_body2(alpha_ref, xl_ref, xr_ref, bias_ref, o_ref, mask_ref):
    _mask_prologue(alpha_ref, mask_ref)
    h = _N // 2
    o_ref[:, :h] = xl_ref[...] * mask_ref[:, :h] + bias_ref[:, :h]
    o_ref[:, h:] = xr_ref[...] * mask_ref[:, h:] + bias_ref[:, h:]


def kernel(x, alpha, bias):
    nt = x.shape[0]
    return pl.pallas_call(
        _fused_body2,
        grid=(nt // _BLK,),
        in_specs=[
            pl.BlockSpec((1, _N), lambda i: (0, 0)),
            pl.BlockSpec((_BLK, _N // 2), lambda i: (i, 0)),
            pl.BlockSpec((_BLK, _N // 2), lambda i: (i, 1)),
            pl.BlockSpec((1, _N), lambda i: (0, 0)),
        ],
        out_specs=pl.BlockSpec((_BLK, _N), lambda i: (i, 0)),
        out_shape=jax.ShapeDtypeStruct((nt, _N), jnp.float32),
        scratch_shapes=[pltpu.VMEM((1, _N), jnp.float32)],
        compiler_params=pltpu.CompilerParams(
            dimension_semantics=("arbitrary",)),
    )(alpha, x, x, bias)


# speedup vs baseline: 1.0097x; 1.0097x over previous
import jax
import jax.numpy as jnp
from jax import lax
from jax.experimental import pallas as pl
from jax.experimental.pallas import tpu as pltpu

_N = 4096
_K = 1024
_T = 4.0
_BLK = 512


def _fused_body(alpha_ref, x_ref, bias_ref, o_ref, mask_ref):
    @pl.when(pl.program_id(0) == 0)
    def _():
        u = alpha_ref[...] * (1.0 / _T)
        mx = jnp.max(u)
        e = jnp.exp(u - mx)
        s = jnp.sum(e)
        bits = jax.lax.bitcast_convert_type(e, jnp.int32)

        def vstep(_, lohi):
            lo, hi = lohi
            mid = lo + (hi - lo) // 2
            ok = jnp.sum((bits >= mid).astype(jnp.int32)) >= _K
            return jnp.where(ok, mid, lo), jnp.where(ok, hi, mid)

        tb, _ = lax.fori_loop(0, 30, vstep,
                              (jnp.int32(0), jnp.int32(1 << 30)))
        n_ge = jnp.sum((bits >= tb).astype(jnp.int32))
        tie = bits == tb
        n_eq = jnp.sum(tie.astype(jnp.int32))
        need = _K - (n_ge - n_eq)

        idx = lax.broadcasted_iota(jnp.int32, (1, _N), 1)

        def istep(_, lohi):
            lo, hi = lohi
            mid = lo + (hi - lo) // 2
            cnt = jnp.sum(jnp.where(tie & (idx >= mid), 1, 0))
            ok = cnt >= need
            return jnp.where(ok, mid, lo), jnp.where(ok, hi, mid)

        c, _ = lax.fori_loop(0, 12, istep,
                             (jnp.int32(0), jnp.int32(_N)))
        keep = (bits > tb) | (tie & (idx >= c))
        mask_ref[...] = jnp.where(keep, e / s, 0.0)

    o_ref[...] = x_ref[...] * mask_ref[...] + bias_ref[...]


def kernel(x, alpha, bias):
    nt = x.shape[0]
    return pl.pallas_call(
        _fused_body,
        grid=(nt // _BLK,),
        in_specs=[
            pl.BlockSpec((1, _N), lambda i: (0, 0)),
            pl.BlockSpec((_BLK, _N), lambda i: (i, 0)),
            pl.BlockSpec((1, _N), lambda i: (0, 0)),
        ],
        out_specs=pl.BlockSpec((_BLK, _N), lambda i: (i, 0)),
        out_shape=jax.ShapeDtypeStruct((nt, _N), jnp.float32),
        scratch_shapes=[pltpu.VMEM((1, _N), jnp.float32)],
        compiler_params=pltpu.CompilerParams(
            dimension_semantics=("arbitrary",)),
    )(alpha, x, bias)
